# baseline (device time: 92899 ns/iter reference)
import jax
import jax.numpy as jnp
from jax import lax
from jax.experimental import pallas as pl
from jax.experimental.pallas import tpu as pltpu

N_DEV = 4
SQ = 256
SKV = 4096
HQ = 8
DH = 128
D = HQ * DH
BLK = 64
NBLK = SKV // BLK
QBLK = SQ // BLK
SEG = 22 * BLK
SKVP = 66 * BLK
SCALE = 0.08838834764831843


def _reorder(a):
    b = a.reshape(NBLK, BLK, D)
    z = jnp.zeros((1, BLK, D), a.dtype)
    return jnp.concatenate([b[0::3], b[1::3], z, b[2::3], z]).reshape(SKVP, D)


def _class_start(c):
    return c * SEG


def kernel(x, Wq, K_ext, V_ext, Wo):
    xb = x[0].astype(jnp.bfloat16)
    wq = Wq.astype(jnp.bfloat16)
    kr = _reorder(K_ext[0].reshape(SKV, D).astype(jnp.bfloat16))
    vr = _reorder(V_ext[0].reshape(SKV, D).astype(jnp.bfloat16))
    wo = Wo.astype(jnp.bfloat16)

    def body(x_ref, wq_ref, k_ref, v_ref, wo_ref, out_ref,
             qbuf, keepbuf, lkeep, conbuf, lcon, rinbuf, lrin,
             r2out, lr2out, r2in, lr2in, cbuf, lloc, ctxbuf,
             qsend, qrecv, r1send, r1recv, r2send, r2recv):
        p = lax.axis_index("i")
        pa = jnp.bitwise_xor(p, 1)
        pb = 3 - p
        pd = 3 - pa

        barrier = pltpu.get_barrier_semaphore()
        for nbr in (pa, pb):
            pl.semaphore_signal(barrier, inc=1, device_id=(nbr,),
                                device_id_type=pl.DeviceIdType.MESH)
        pl.semaphore_wait(barrier, 2)

        pending = []

        def rdma(src, dst, ssem, rsem, dev):
            return pltpu.make_async_remote_copy(
                src_ref=src, dst_ref=dst, send_sem=ssem, recv_sem=rsem,
                device_id=(dev,), device_id_type=pl.DeviceIdType.MESH)

        def send(src, dst, ssem, rsem, dev):
            r = rdma(src, dst, ssem, rsem, dev)
            r.start()
            pending.append(r)

        qbuf[0] = (jnp.dot(x_ref[...], wq_ref[...],
                           preferred_element_type=jnp.float32)
                   * SCALE).astype(jnp.bfloat16)

        send(qbuf.at[0], qbuf.at[1], qsend.at[0], qrecv.at[0], pa)
        send(qbuf.at[0], qbuf.at[2], qsend.at[1], qrecv.at[1], pb)

        def contrib(o, slot):
            for qb in range(QBLK):
                qbg = 4 * o + qb
                r = lax.rem(3 - lax.rem(qbg + p, 3), 3)
                seg_start = _class_start(r)
                npad = (jnp.minimum(r, 1) * BLK).astype(jnp.float32)
                diag_start = _class_start(lax.rem(qbg, 3)) + (qbg // 3) * BLK
                diagf = jnp.where((p == 0) & (lax.rem(qbg, 3) != 0),
                                  1.0, 0.0).astype(jnp.bfloat16)
                b0f = jnp.where((p == 0) & (r != 0), 1.0, 0.0).astype(jnp.bfloat16)
                rsl = pl.ds(qb * BLK, BLK)
                for hd in range(HQ):
                    sl = pl.ds(hd * DH, DH)
                    qh = qbuf[slot, rsl, sl]
                    kseg = k_ref[pl.ds(seg_start, SEG), sl]
                    s = lax.dot_general(qh, kseg, (((1,), (1,)), ((), ())),
                                        preferred_element_type=jnp.float32)
                    pe = jnp.exp(s.astype(jnp.bfloat16))
                    lp = jnp.sum(pe, axis=1, keepdims=True,
                                 dtype=jnp.float32) - npad
                    acc = lax.dot_general(pe, v_ref[pl.ds(seg_start, SEG), sl],
                                          (((1,), (0,)), ((), ())),
                                          preferred_element_type=jnp.float32)
                    kd = k_ref[pl.ds(diag_start, BLK), sl]
                    sd = lax.dot_general(qh, kd, (((1,), (1,)), ((), ())),
                                         preferred_element_type=jnp.float32)
                    ped = jnp.exp(sd.astype(jnp.bfloat16)) * diagf
                    k0 = k_ref[pl.ds(0, BLK), sl]
                    s0 = lax.dot_general(qh, k0, (((1,), (1,)), ((), ())),
                                         preferred_element_type=jnp.float32)
                    pe0 = jnp.exp(s0.astype(jnp.bfloat16)) * b0f
                    lp = lp + jnp.sum(ped, axis=1, keepdims=True,
                                      dtype=jnp.float32) \
                            + jnp.sum(pe0, axis=1, keepdims=True,
                                      dtype=jnp.float32)
                    acc = acc + lax.dot_general(
                        ped, v_ref[pl.ds(diag_start, BLK), sl],
                        (((1,), (0,)), ((), ())),
                        preferred_element_type=jnp.float32) \
                        + lax.dot_general(
                        pe0, v_ref[pl.ds(0, BLK), sl],
                        (((1,), (0,)), ((), ())),
                        preferred_element_type=jnp.float32)
                    cbuf[rsl, sl] = acc
                    lloc[rsl, hd:hd + 1] = lp

        contrib(p, 0)
        keepbuf[0] = cbuf[...]
        lkeep[0] = lloc[...]

        rdma(qbuf.at[2], qbuf.at[2], qsend.at[1], qrecv.at[1], pb).wait_recv()
        send(qbuf.at[2], qbuf.at[3], qsend.at[2], qrecv.at[2], pa)
        contrib(pb, 2)
        keepbuf[1] = cbuf[...]
        lkeep[1] = lloc[...]

        rdma(qbuf.at[1], qbuf.at[1], qsend.at[0], qrecv.at[0], pa).wait_recv()
        contrib(pa, 1)
        conbuf[0] = cbuf[...].astype(jnp.bfloat16)
        lcon[0] = lloc[...]
        send(conbuf.at[0], rinbuf.at[0], r1send.at[0], r1recv.at[0], pa)
        send(lcon.at[0], lrin.at[0], r1send.at[1], r1recv.at[1], pa)

        rdma(qbuf.at[3], qbuf.at[3], qsend.at[2], qrecv.at[2], pa).wait_recv()
        contrib(pd, 3)
        conbuf[1] = cbuf[...].astype(jnp.bfloat16)
        lcon[1] = lloc[...]
        send(conbuf.at[1], rinbuf.at[1], r1send.at[2], r1recv.at[2], pa)
        send(lcon.at[1], lrin.at[1], r1send.at[3], r1recv.at[3], pa)

        rdma(rinbuf.at[1], rinbuf.at[1], r1send.at[2], r1recv.at[2], pa).wait_recv()
        rdma(lrin.at[1], lrin.at[1], r1send.at[3], r1recv.at[3], pa).wait_recv()
        r2out[...] = (keepbuf[1] + rinbuf[1].astype(jnp.float32)
                      ).astype(jnp.bfloat16)
        lr2out[...] = lkeep[1] + lrin[1]
        send(r2out, r2in, r2send.at[0], r2recv.at[0], pb)
        send(lr2out, lr2in, r2send.at[1], r2recv.at[1], pb)

        rdma(rinbuf.at[0], rinbuf.at[0], r1send.at[0], r1recv.at[0], pa).wait_recv()
        rdma(lrin.at[0], lrin.at[0], r1send.at[1], r1recv.at[1], pa).wait_recv()

        rdma(r2in, r2in, r2send.at[0], r2recv.at[0], pb).wait_recv()
        rdma(lr2in, lr2in, r2send.at[1], r2recv.at[1], pb).wait_recv()

        for hd in range(HQ):
            sl = pl.ds(hd * DH, DH)
            acc_t = (keepbuf[0, :, sl] + rinbuf[0, :, sl].astype(jnp.float32)
                     + r2in[:, sl].astype(jnp.float32))
            l_t = lkeep[0, :, hd:hd + 1] + lrin[0, :, hd:hd + 1] \
                + lr2in[:, hd:hd + 1]
            ctxbuf[:, sl] = (acc_t / l_t).astype(jnp.bfloat16)
        out_ref[...] = lax.dot_general(ctxbuf[...], wo_ref[...],
                                       (((1,), (0,)), ((), ())),
                                       preferred_element_type=jnp.float32)

        for r in pending:
            r.wait_send()

    out = pl.pallas_call(
        body,
        out_shape=jax.ShapeDtypeStruct((SQ, D), jnp.float32),
        in_specs=[pl.BlockSpec(memory_space=pltpu.VMEM)] * 5,
        out_specs=pl.BlockSpec(memory_space=pltpu.VMEM),
        scratch_shapes=[
            pltpu.VMEM((4, SQ, D), jnp.bfloat16),
            pltpu.VMEM((2, SQ, D), jnp.float32),
            pltpu.VMEM((2, SQ, HQ), jnp.float32),
            pltpu.VMEM((2, SQ, D), jnp.bfloat16),
            pltpu.VMEM((2, SQ, HQ), jnp.float32),
            pltpu.VMEM((2, SQ, D), jnp.bfloat16),
            pltpu.VMEM((2, SQ, HQ), jnp.float32),
            pltpu.VMEM((SQ, D), jnp.bfloat16),
            pltpu.VMEM((SQ, HQ), jnp.float32),
            pltpu.VMEM((SQ, D), jnp.bfloat16),
            pltpu.VMEM((SQ, HQ), jnp.float32),
            pltpu.VMEM((SQ, D), jnp.float32),
            pltpu.VMEM((SQ, HQ), jnp.float32),
            pltpu.VMEM((SQ, D), jnp.bfloat16),
            pltpu.SemaphoreType.DMA((3,)),
            pltpu.SemaphoreType.DMA((3,)),
            pltpu.SemaphoreType.DMA((4,)),
            pltpu.SemaphoreType.DMA((4,)),
            pltpu.SemaphoreType.DMA((2,)),
            pltpu.SemaphoreType.DMA((2,)),
        ],
        compiler_params=pltpu.CompilerParams(collective_id=0),
    )(xb, wq, kr, vr, wo)
    return out[None]
